# Initial kernel scaffold; baseline (speedup 1.0000x reference)
import jax
import jax.numpy as jnp
from jax.experimental import pallas as pl


def kernel(x, edge_index, W1, b1, W2, b2):
    raise NotImplementedError("write your pallas kernel here")


# speedup vs baseline: 2.6357x; 2.6357x over previous
import functools

import jax
import jax.numpy as jnp
from jax import lax
from jax.experimental import pallas as pl
from jax.experimental.pallas import tpu as pltpu
from jax.experimental.pallas import tpu_sc as plsc

N = 10000
E = 160000
D_IN = 256
D_HID = 512
D_OUT = 512

NC = 2
NS = 16
NW = NC * NS
LANES = 16
NB = 64
RPB = 160
ACC_R = 168
CHUNK = 128
CAP = 3584
NCHMAX = CAP // CHUNK

_mesh = plsc.VectorSubcoreMesh(
    core_axis_name="c", subcore_axis_name="s", num_cores=NC, num_subcores=NS
)

_f32 = jnp.float32
_i32 = jnp.int32


def _zero_1d(ref, n):
  z = jnp.zeros((LANES,), _f32)
  for j in range(n // LANES):
    ref[pl.ds(j * LANES, LANES)] = z


def _sc_args(with_feat, with_r):
  scratch = [
      pltpu.VMEM((CHUNK + LANES,), _i32),
      pltpu.VMEM((CHUNK + LANES,), _i32),
      pltpu.VMEM((CHUNK + LANES,), _i32),
  ]
  if with_feat:
    scratch += [
        pltpu.VMEM((CHUNK, D_IN), _f32),
        pltpu.VMEM((ACC_R, D_IN), _f32),
        pltpu.SemaphoreType.DMA,
    ]
  if with_r:
    scratch += [pltpu.VMEM((10032,), _f32)]
  if with_r or not with_feat:
    scratch += [pltpu.VMEM((272,), _f32)]
  return scratch


def _make_sc(with_feat, with_r):
  def body(*args):
    a = list(args)
    feat_hbm = a.pop(0) if with_feat else None
    dinv_hbm = a.pop(0) if with_r else None
    src_hbm, ldst_hbm, cnt_hbm = a.pop(0), a.pop(0), a.pop(0)
    out_hbm = a.pop(0) if with_feat else None
    rout_hbm = a.pop(0) if with_r or not with_feat else None
    srcv, ldv, cntv = a.pop(0), a.pop(0), a.pop(0)
    if with_feat:
      buf, acc, sem = a.pop(0), a.pop(0), a.pop(0)
    if with_r:
      dinv_v = a.pop(0)
    if with_r or not with_feat:
      racc = a.pop(0)

    cid = lax.axis_index("c")
    sid = lax.axis_index("s")
    wid = sid * NC + cid

    if with_r:
      pltpu.sync_copy(dinv_hbm, dinv_v)

    lane0 = lax.iota(_i32, LANES) == 0
    one16 = jnp.where(lane0, 1.0, 0.0).astype(_f32)

    for half in range(2):
      b = wid * 2 + half

      if with_feat:
        @pl.loop(0, ACC_R)
        def _(rr):
          for j in range(D_IN // LANES):
            acc[rr, pl.ds(j * LANES, LANES)] = jnp.zeros((LANES,), _f32)
      if with_r or not with_feat:
        _zero_1d(racc, 272)

      pltpu.sync_copy(cnt_hbm.at[b], cntv.at[pl.ds(0, CHUNK)])
      cnt = cntv[pl.ds(0, LANES)][0]
      nch = (cnt + CHUNK - 1) // CHUNK

      def chunk_body(c, _):
        pltpu.sync_copy(ldst_hbm.at[b, c], ldv.at[pl.ds(0, CHUNK)])
        if with_feat or with_r:
          pltpu.sync_copy(src_hbm.at[b, c], srcv.at[pl.ds(0, CHUNK)])
        if with_feat:
          pltpu.async_copy(
              feat_hbm.at[srcv.at[pl.ds(0, CHUNK)]], buf, sem).wait()

        def edge_body(i, _):
          ld = ldv[pl.ds(i, LANES)][0]
          if with_feat:
            for j in range(D_IN // LANES):
              sl = pl.ds(j * LANES, LANES)
              acc[ld, sl] = acc[ld, sl] + buf[i, sl]
          rsl = pl.ds(ld, LANES)
          if with_r:
            s = srcv[pl.ds(i, LANES)][0]
            dval = dinv_v[pl.ds(s, LANES)][0]
            racc[rsl] = racc[rsl] + jnp.where(lane0, dval, 0.0)
          elif not with_feat:
            racc[rsl] = racc[rsl] + one16
          return 0

        lax.fori_loop(0, CHUNK, edge_body, 0)
        return 0

      lax.fori_loop(0, nch, chunk_body, 0)

      if with_feat:
        pltpu.sync_copy(acc.at[pl.ds(0, RPB)], out_hbm.at[b])
      if with_r or not with_feat:
        pltpu.sync_copy(racc.at[pl.ds(0, 256)], rout_hbm.at[b])

  out_type = []
  if with_feat:
    out_type.append(jax.ShapeDtypeStruct((NB, RPB, D_IN), _f32))
  if with_r or not with_feat:
    out_type.append(jax.ShapeDtypeStruct((NB, 256), _f32))
  return pl.kernel(
      body,
      out_type=out_type if len(out_type) > 1 else out_type[0],
      mesh=_mesh,
      scratch_types=_sc_args(with_feat, with_r),
  )


_k_deg = _make_sc(False, False)
_k_agg_r = _make_sc(True, True)
_k_agg = _make_sc(True, False)


_ROWS = 1000


def _k2_body(deg_ref, x_ref, xt_ref, dinv_ref):
  dinv = lax.rsqrt(deg_ref[...] + 1.0)
  xt_ref[...] = x_ref[...] * dinv
  dinv_ref[...] = dinv


def _k4_body(p_ref, xt_ref, dinv_ref, yt_ref):
  d = dinv_ref[...]
  yt_ref[...] = (p_ref[...] + xt_ref[...]) * (d * d)


def _k6_body(q_ref, yt_ref, dinv_ref, r_ref, c_ref, out_ref):
  d = dinv_ref[...]
  z = (q_ref[...] + yt_ref[...]) * d
  cval = c_ref[...]
  w12 = cval[:D_IN]
  c1 = cval[D_IN]
  b2 = cval[D_IN + 8]
  r = (r_ref[...] + d) * d
  out_ref[...] = (
      jnp.dot(z, w12, preferred_element_type=_f32)
      + r * c1[None, :]
      + b2[None, :]
  )


def _k0_body(a0_ref, w2_ref, c_ref):
  c_ref[...] = jnp.dot(a0_ref[...], w2_ref[...], preferred_element_type=_f32)


def _row_spec(cols):
  return pl.BlockSpec((_ROWS, cols), lambda i: (i, 0))


_k2 = pl.pallas_call(
    _k2_body,
    grid=(N // _ROWS,),
    in_specs=[_row_spec(1), _row_spec(D_IN)],
    out_specs=[_row_spec(D_IN), _row_spec(1)],
    out_shape=[
        jax.ShapeDtypeStruct((N, D_IN), _f32),
        jax.ShapeDtypeStruct((N, 1), _f32),
    ],
)

_k4 = pl.pallas_call(
    _k4_body,
    grid=(N // _ROWS,),
    in_specs=[_row_spec(D_IN), _row_spec(D_IN), _row_spec(1)],
    out_specs=_row_spec(D_IN),
    out_shape=jax.ShapeDtypeStruct((N, D_IN), _f32),
)

_k6 = pl.pallas_call(
    _k6_body,
    grid=(N // _ROWS,),
    in_specs=[
        _row_spec(D_IN),
        _row_spec(D_IN),
        _row_spec(1),
        _row_spec(1),
        pl.BlockSpec((D_IN + LANES, D_HID), lambda i: (0, 0)),
    ],
    out_specs=_row_spec(D_OUT),
    out_shape=jax.ShapeDtypeStruct((N, D_OUT), _f32),
)

_k0 = pl.pallas_call(
    _k0_body,
    out_shape=jax.ShapeDtypeStruct((D_IN + 8, D_HID), _f32),
)


def kernel(x, edge_index, W1, b1, W2, b2):
  ei = edge_index.astype(_i32)
  src, dst = ei[0], ei[1]

  owner = dst // RPB
  order = jnp.argsort(owner)
  src_s, dst_s = src[order], dst[order]
  owner_s = owner[order]
  starts = jnp.searchsorted(owner_s, jnp.arange(NB, dtype=_i32))
  cnts = jnp.diff(jnp.append(starts, E)).astype(_i32)
  rank = jnp.arange(E, dtype=_i32) - starts[owner_s].astype(_i32)
  dest = jnp.where(rank < CAP, owner_s * CAP + rank, NB * CAP)
  psrc = jnp.zeros((NB * CAP,), _i32).at[dest].set(src_s, mode="drop")
  pldst = jnp.full((NB * CAP,), RPB, _i32).at[dest].set(
      dst_s % RPB, mode="drop")
  psrc = psrc.reshape(NB, NCHMAX, CHUNK)
  pldst = pldst.reshape(NB, NCHMAX, CHUNK)
  cnt2d = jnp.zeros((NB, CHUNK), _i32).at[:, 0].set(cnts)

  deg2d = _k_deg(psrc, pldst, cnt2d)
  deg = deg2d[:, :RPB].reshape(NB * RPB)[:N, None]

  xt, dinv = _k2(deg, x)
  dinv_flat = jnp.concatenate([dinv[:, 0], jnp.zeros((10032 - N,), _f32)])

  p3, r2 = _k_agg_r(xt, dinv_flat, psrc, pldst, cnt2d)
  p = p3.reshape(NB * RPB, D_IN)[:N]
  r = r2[:, :RPB].reshape(NB * RPB)[:N, None]

  yt = _k4(p, xt, dinv)

  q3 = _k_agg(yt, psrc, pldst, cnt2d)
  q = q3.reshape(NB * RPB, D_IN)[:N]

  a0 = jnp.concatenate([W1, b1[None, :], jnp.zeros((7, D_HID), _f32)], axis=0)
  c = _k0(a0, W2)
  c2 = jnp.concatenate([c, b2[None, :], jnp.zeros((7, D_OUT), _f32)], axis=0)
  return _k6(q, yt, dinv, r, c2)


# speedup vs baseline: 2.8389x; 1.0771x over previous
import functools

import jax
import jax.numpy as jnp
from jax import lax
from jax.experimental import pallas as pl
from jax.experimental.pallas import tpu as pltpu
from jax.experimental.pallas import tpu_sc as plsc

N = 10000
E = 160000
D_IN = 256
D_HID = 512
D_OUT = 512

NC = 2
NS = 16
NW = NC * NS
LANES = 16
NB = 64
RPB = 160
ACC_R = 168
CHUNK = 128
CAP = 3584
NCHMAX = CAP // CHUNK

_mesh = plsc.VectorSubcoreMesh(
    core_axis_name="c", subcore_axis_name="s", num_cores=NC, num_subcores=NS
)

_f32 = jnp.float32
_i32 = jnp.int32


def _zero_1d(ref, n):
  z = jnp.zeros((LANES,), _f32)
  for j in range(n // LANES):
    ref[pl.ds(j * LANES, LANES)] = z


def _sc_args(with_feat, with_r):
  scratch = [
      pltpu.VMEM((CHUNK + LANES,), _i32),
      pltpu.VMEM((CHUNK + LANES,), _i32),
      pltpu.VMEM((CHUNK + LANES,), _i32),
  ]
  if with_feat:
    scratch += [
        pltpu.VMEM((CHUNK, D_IN), _f32),
        pltpu.VMEM((ACC_R, D_IN), _f32),
        pltpu.SemaphoreType.DMA,
    ]
  if with_r:
    scratch += [pltpu.VMEM((10032,), _f32)]
  if with_r or not with_feat:
    scratch += [pltpu.VMEM((272,), _f32)]
  return scratch


def _make_sc(with_feat, with_r):
  def body(*args):
    a = list(args)
    feat_hbm = a.pop(0) if with_feat else None
    dinv_hbm = a.pop(0) if with_r else None
    src_hbm, ldst_hbm, cnt_hbm = a.pop(0), a.pop(0), a.pop(0)
    out_hbm = a.pop(0) if with_feat else None
    rout_hbm = a.pop(0) if with_r or not with_feat else None
    srcv, ldv, cntv = a.pop(0), a.pop(0), a.pop(0)
    if with_feat:
      buf, acc, sem = a.pop(0), a.pop(0), a.pop(0)
    if with_r:
      dinv_v = a.pop(0)
    if with_r or not with_feat:
      racc = a.pop(0)

    cid = lax.axis_index("c")
    sid = lax.axis_index("s")
    wid = sid * NC + cid

    if with_r:
      pltpu.sync_copy(dinv_hbm, dinv_v)

    lane0 = lax.iota(_i32, LANES) == 0
    one16 = jnp.where(lane0, 1.0, 0.0).astype(_f32)

    for half in range(2):
      b = wid * 2 + half

      if with_feat:
        @pl.loop(0, ACC_R)
        def _(rr):
          for j in range(D_IN // LANES):
            acc[rr, pl.ds(j * LANES, LANES)] = jnp.zeros((LANES,), _f32)
      if with_r or not with_feat:
        _zero_1d(racc, 272)

      pltpu.sync_copy(cnt_hbm.at[b], cntv.at[pl.ds(0, CHUNK)])
      cnt = cntv[pl.ds(0, LANES)][0]
      nch = (cnt + CHUNK - 1) // CHUNK

      def chunk_body(c, _):
        pltpu.sync_copy(ldst_hbm.at[b, c], ldv.at[pl.ds(0, CHUNK)])
        if with_feat or with_r:
          pltpu.sync_copy(src_hbm.at[b, c], srcv.at[pl.ds(0, CHUNK)])
        if with_feat:
          pltpu.async_copy(
              feat_hbm.at[srcv.at[pl.ds(0, CHUNK)]], buf, sem).wait()

        U = 8

        def edge_body(i, _):
          base = i * U
          ldq = ldv[pl.ds(base, LANES)]
          if with_r:
            sq = srcv[pl.ds(base, LANES)]
          for u in range(U):
            ld = ldq[u]
            if with_feat:
              for j in range(D_IN // LANES):
                sl = pl.ds(j * LANES, LANES)
                acc[ld, sl] = acc[ld, sl] + buf[base + u, sl]
            rsl = pl.ds(ld, LANES)
            if with_r:
              dval = dinv_v[pl.ds(sq[u], LANES)][0]
              racc[rsl] = racc[rsl] + jnp.where(lane0, dval, 0.0)
            elif not with_feat:
              racc[rsl] = racc[rsl] + one16
          return 0

        lax.fori_loop(0, CHUNK // U, edge_body, 0)
        return 0

      lax.fori_loop(0, nch, chunk_body, 0)

      if with_feat:
        pltpu.sync_copy(acc.at[pl.ds(0, RPB)], out_hbm.at[b])
      if with_r or not with_feat:
        pltpu.sync_copy(racc.at[pl.ds(0, 256)], rout_hbm.at[b])

  out_type = []
  if with_feat:
    out_type.append(jax.ShapeDtypeStruct((NB, RPB, D_IN), _f32))
  if with_r or not with_feat:
    out_type.append(jax.ShapeDtypeStruct((NB, 256), _f32))
  return pl.kernel(
      body,
      out_type=out_type if len(out_type) > 1 else out_type[0],
      mesh=_mesh,
      scratch_types=_sc_args(with_feat, with_r),
  )


_k_deg = _make_sc(False, False)
_k_agg_r = _make_sc(True, True)
_k_agg = _make_sc(True, False)


_ROWS = 1000


def _k2_body(deg_ref, x_ref, xt_ref, dinv_ref):
  dinv = lax.rsqrt(deg_ref[...] + 1.0)
  xt_ref[...] = x_ref[...] * dinv
  dinv_ref[...] = dinv


def _k4_body(p_ref, xt_ref, dinv_ref, yt_ref):
  d = dinv_ref[...]
  yt_ref[...] = (p_ref[...] + xt_ref[...]) * (d * d)


def _k6_body(q_ref, yt_ref, dinv_ref, r_ref, c_ref, out_ref):
  d = dinv_ref[...]
  z = (q_ref[...] + yt_ref[...]) * d
  cval = c_ref[...]
  w12 = cval[:D_IN]
  c1 = cval[D_IN]
  b2 = cval[D_IN + 8]
  r = (r_ref[...] + d) * d
  out_ref[...] = (
      jnp.dot(z, w12, preferred_element_type=_f32)
      + r * c1[None, :]
      + b2[None, :]
  )


def _k0_body(a0_ref, w2_ref, c_ref):
  c_ref[...] = jnp.dot(a0_ref[...], w2_ref[...], preferred_element_type=_f32)


def _row_spec(cols):
  return pl.BlockSpec((_ROWS, cols), lambda i: (i, 0))


_k2 = pl.pallas_call(
    _k2_body,
    grid=(N // _ROWS,),
    in_specs=[_row_spec(1), _row_spec(D_IN)],
    out_specs=[_row_spec(D_IN), _row_spec(1)],
    out_shape=[
        jax.ShapeDtypeStruct((N, D_IN), _f32),
        jax.ShapeDtypeStruct((N, 1), _f32),
    ],
)

_k4 = pl.pallas_call(
    _k4_body,
    grid=(N // _ROWS,),
    in_specs=[_row_spec(D_IN), _row_spec(D_IN), _row_spec(1)],
    out_specs=_row_spec(D_IN),
    out_shape=jax.ShapeDtypeStruct((N, D_IN), _f32),
)

_k6 = pl.pallas_call(
    _k6_body,
    grid=(N // _ROWS,),
    in_specs=[
        _row_spec(D_IN),
        _row_spec(D_IN),
        _row_spec(1),
        _row_spec(1),
        pl.BlockSpec((D_IN + LANES, D_HID), lambda i: (0, 0)),
    ],
    out_specs=_row_spec(D_OUT),
    out_shape=jax.ShapeDtypeStruct((N, D_OUT), _f32),
)

_k0 = pl.pallas_call(
    _k0_body,
    out_shape=jax.ShapeDtypeStruct((D_IN + 8, D_HID), _f32),
)


def kernel(x, edge_index, W1, b1, W2, b2):
  ei = edge_index.astype(_i32)
  src, dst = ei[0], ei[1]

  owner = dst // RPB
  order = jnp.argsort(owner)
  src_s, dst_s = src[order], dst[order]
  owner_s = owner[order]
  starts = jnp.searchsorted(owner_s, jnp.arange(NB, dtype=_i32))
  cnts = jnp.diff(jnp.append(starts, E)).astype(_i32)
  rank = jnp.arange(E, dtype=_i32) - starts[owner_s].astype(_i32)
  dest = jnp.where(rank < CAP, owner_s * CAP + rank, NB * CAP)
  psrc = jnp.zeros((NB * CAP,), _i32).at[dest].set(src_s, mode="drop")
  pldst = jnp.full((NB * CAP,), RPB, _i32).at[dest].set(
      dst_s % RPB, mode="drop")
  psrc = psrc.reshape(NB, NCHMAX, CHUNK)
  pldst = pldst.reshape(NB, NCHMAX, CHUNK)
  cnt2d = jnp.zeros((NB, CHUNK), _i32).at[:, 0].set(cnts)

  deg2d = _k_deg(psrc, pldst, cnt2d)
  deg = deg2d[:, :RPB].reshape(NB * RPB)[:N, None]

  xt, dinv = _k2(deg, x)
  dinv_flat = jnp.concatenate([dinv[:, 0], jnp.zeros((10032 - N,), _f32)])

  p3, r2 = _k_agg_r(xt, dinv_flat, psrc, pldst, cnt2d)
  p = p3.reshape(NB * RPB, D_IN)[:N]
  r = r2[:, :RPB].reshape(NB * RPB)[:N, None]

  yt = _k4(p, xt, dinv)

  q3 = _k_agg(yt, psrc, pldst, cnt2d)
  q = q3.reshape(NB * RPB, D_IN)[:N]

  a0 = jnp.concatenate([W1, b1[None, :], jnp.zeros((7, D_HID), _f32)], axis=0)
  c = _k0(a0, W2)
  c2 = jnp.concatenate([c, b2[None, :], jnp.zeros((7, D_OUT), _f32)], axis=0)
  return _k6(q, yt, dinv, r, c2)
